# baseline (speedup 1.0000x reference)
import functools

import jax
import jax.numpy as jnp
from jax import lax
from jax.experimental import pallas as pl
from jax.experimental.pallas import tpu as pltpu
from jax.experimental.pallas import tpu_sc as plsc

N_NODES = 10000
N_EDGES = 320000
D = 128

NC = 2
NS = 16
NW = NC * NS
LANE = 32

N_PAD = 10112
E_PAD = 327680
ROWS_TOTAL = E_PAD // LANE
ROWS_PW = ROWS_TOTAL // NW
GCH = 8
NBUF = 4
FAST_CID = 1
FAST_ROWS = 400
SLOW_ROWS = 240
STRIPE = N_PAD // NS
DEG_W = 128


def _sc_mesh():
    return plsc.VectorSubcoreMesh(core_axis_name="c", subcore_axis_name="s",
                                  num_cores=NC, num_subcores=NS)


def _zero_stripe(z_hbm, stage, sh, sid):
    pltpu.sync_copy(z_hbm, stage)
    full, rem = divmod(STRIPE, LANE)
    for k in range(full):
        pltpu.sync_copy(stage, sh.at[pl.ds(sid * STRIPE + k * LANE, LANE)])
    if rem:
        pltpu.sync_copy(stage.at[pl.ds(0, rem)],
                        sh.at[pl.ds(sid * STRIPE + full * LANE, rem)])


@functools.cache
def _sc_agg():
    def body(h_hbm, src_hbm, dst_hbm, z_hbm, agg_out,
             idx_v, rows, sg0, sg1, sg2, sg3, ss0, ss1, ss2, ss3, agg_sh):
        cid = lax.axis_index("c")
        sid = lax.axis_index("s")
        wid = cid * NS + sid

        _zero_stripe(z_hbm, rows.at[0], agg_sh, sid)
        plsc.subcore_barrier()

        base = jnp.where(cid == FAST_CID,
                         sid * FAST_ROWS,
                         NS * FAST_ROWS + sid * SLOW_ROWS)
        nchunks = jnp.where(cid == FAST_CID, FAST_ROWS // GCH,
                            SLOW_ROWS // GCH)
        sg = (sg0, sg1, sg2, sg3)
        ss = (ss0, ss1, ss2, ss3)
        LOOK = NBUF - 1

        def chunk(c, carry):
            pltpu.sync_copy(src_hbm.at[pl.ds(base + c * GCH, GCH)],
                            idx_v.at[0])
            pltpu.sync_copy(dst_hbm.at[pl.ds(base + c * GCH, GCH)],
                            idx_v.at[1])

            def gather(r):
                return pltpu.async_copy(h_hbm.at[idx_v.at[0, r]],
                                        rows.at[r % NBUF], sg[r % NBUF])

            gat = {r: gather(r) for r in range(LOOK)}
            sca = {}
            for r in range(GCH):
                gat[r].wait()
                nxt = r + LOOK
                if nxt < GCH:
                    if r >= 1:
                        sca[r - 1].wait()
                    gat[nxt] = gather(nxt)
                sca[r] = pltpu.async_copy(rows.at[r % NBUF],
                                          agg_sh.at[idx_v.at[1, r]],
                                          ss[r % NBUF], add=True)
            for r in range(max(0, GCH - LOOK - 1), GCH):
                sca[r].wait()
            return carry

        lax.fori_loop(0, nchunks, chunk, 0)

        plsc.subcore_barrier()
        pltpu.sync_copy(agg_sh.at[pl.ds(sid * STRIPE, STRIPE)],
                        agg_out.at[cid, pl.ds(sid * STRIPE, STRIPE)])

    return pl.kernel(
        body,
        out_type=jax.ShapeDtypeStruct((NC, N_PAD, D), jnp.float32),
        mesh=_sc_mesh(),
        scratch_types=[
            pltpu.VMEM((2, GCH, LANE), jnp.int32),
            pltpu.VMEM((NBUF, LANE, D), jnp.float32),
            pltpu.SemaphoreType.DMA,
            pltpu.SemaphoreType.DMA,
            pltpu.SemaphoreType.DMA,
            pltpu.SemaphoreType.DMA,
            pltpu.SemaphoreType.DMA,
            pltpu.SemaphoreType.DMA,
            pltpu.SemaphoreType.DMA,
            pltpu.SemaphoreType.DMA,
            pltpu.VMEM_SHARED((N_PAD, D), jnp.float32),
        ],
    )


@functools.cache
def _sc_deg():
    def body(dst_hbm, z_hbm, ones_hbm, deg_out, idx_v, ones_v, sem, deg_sh):
        cid = lax.axis_index("c")
        sid = lax.axis_index("s")
        wid = cid * NS + sid

        _zero_stripe(z_hbm, ones_v, deg_sh, sid)
        pltpu.sync_copy(ones_hbm, ones_v)
        plsc.subcore_barrier()

        base = wid * ROWS_PW

        def chunk(c, carry):
            pltpu.sync_copy(dst_hbm.at[pl.ds(base + c * GCH, GCH)],
                            idx_v.at[0])
            sca = [pltpu.async_copy(ones_v, deg_sh.at[idx_v.at[0, r]],
                                    sem, add=True)
                   for r in range(GCH)]
            for s in sca:
                s.wait()
            return carry

        lax.fori_loop(0, ROWS_PW // GCH, chunk, 0)

        plsc.subcore_barrier()
        pltpu.sync_copy(deg_sh.at[pl.ds(sid * STRIPE, STRIPE)],
                        deg_out.at[cid, pl.ds(sid * STRIPE, STRIPE)])

    return pl.kernel(
        body,
        out_type=jax.ShapeDtypeStruct((NC, N_PAD, DEG_W), jnp.float32),
        mesh=_sc_mesh(),
        scratch_types=[
            pltpu.VMEM((1, GCH, LANE), jnp.int32),
            pltpu.VMEM((LANE, DEG_W), jnp.float32),
            pltpu.SemaphoreType.DMA,
            pltpu.VMEM_SHARED((N_PAD, DEG_W), jnp.float32),
        ],
    )


ROW_BLK = 1264


_DN = (((1,), (1,)), ((), ()))


def _tc_self_body(x_ref, ws_ref, b_ref, out_ref):
    out_ref[...] = lax.dot_general(x_ref[...], ws_ref[...], _DN,
                                   preferred_element_type=jnp.float32) + b_ref[...]


def _tc1_body(s_ref, aggp_ref, degp_ref, wn_ref, out_ref):
    agg = aggp_ref[0] + aggp_ref[1]
    deg = degp_ref[0, :, 0:1] + degp_ref[1, :, 0:1]
    hn = agg / jnp.maximum(deg, 1.0)
    h = s_ref[...] + lax.dot_general(hn, wn_ref[...], _DN,
                                     preferred_element_type=jnp.float32)
    out_ref[...] = jnp.maximum(h, 0.0)


def _tc2_body(s_ref, aggp_ref, degp_ref, wn_ref, w3_ref, b3_ref, out_ref):
    agg = aggp_ref[0] + aggp_ref[1]
    deg = degp_ref[0, :, 0:1] + degp_ref[1, :, 0:1]
    hn = agg / jnp.maximum(deg, 1.0)
    h2 = s_ref[...] + lax.dot_general(hn, wn_ref[...], _DN,
                                      preferred_element_type=jnp.float32)
    out_ref[...] = lax.dot_general(h2, w3_ref[...], _DN,
                                   preferred_element_type=jnp.float32) + b3_ref[...]


def _row_specs():
    row = pl.BlockSpec((ROW_BLK, D), lambda i: (i, 0))
    aggp = pl.BlockSpec((NC, ROW_BLK, D), lambda i: (0, i, 0))
    degp = pl.BlockSpec((NC, ROW_BLK, DEG_W), lambda i: (0, i, 0))
    w = pl.BlockSpec((D, D), lambda i: (0, 0))
    b = pl.BlockSpec((1, D), lambda i: (0, 0))
    return row, aggp, degp, w, b


def _tc_self(x, ws, b):
    row, _, _, w, bias = _row_specs()
    return pl.pallas_call(
        _tc_self_body,
        grid=(N_PAD // ROW_BLK,),
        in_specs=[row, w, bias],
        out_specs=row,
        out_shape=jax.ShapeDtypeStruct((N_PAD, D), jnp.float32),
    )(x, ws, b)


def _tc1(s, aggp, degp, wn):
    row, aggp_s, degp_s, w, _ = _row_specs()
    return pl.pallas_call(
        _tc1_body,
        grid=(N_PAD // ROW_BLK,),
        in_specs=[row, aggp_s, degp_s, w],
        out_specs=row,
        out_shape=jax.ShapeDtypeStruct((N_PAD, D), jnp.float32),
    )(s, aggp, degp, wn)


def _tc2(s, aggp, degp, wn, w3, b3):
    row, aggp_s, degp_s, w, bias = _row_specs()
    return pl.pallas_call(
        _tc2_body,
        grid=(N_PAD // ROW_BLK,),
        in_specs=[row, aggp_s, degp_s, w, w, bias],
        out_specs=row,
        out_shape=jax.ShapeDtypeStruct((N_PAD, D), jnp.float32),
    )(s, aggp, degp, wn, w3, b3)


@jax.jit
def kernel(x, edge_index, W1_self, W1_neigh, b1, W2_self, W2_neigh, b2,
           W3, b3):
    src = edge_index[0].astype(jnp.int32)
    dst = edge_index[1].astype(jnp.int32)
    pad = E_PAD - N_EDGES
    src_p = jnp.concatenate([src, jnp.zeros((pad,), jnp.int32)])
    dst_p = jnp.concatenate([dst, jnp.full((pad,), N_NODES, jnp.int32)])
    src2 = src_p.reshape(ROWS_TOTAL, LANE)
    dst2 = dst_p.reshape(ROWS_TOTAL, LANE)

    x_p = jnp.pad(x, ((0, N_PAD - N_NODES), (0, 0)))
    z128 = jnp.zeros((LANE, D), jnp.float32)
    ones = jnp.ones((LANE, DEG_W), jnp.float32)

    deg_p = _sc_deg()(dst2, z128, ones)
    agg1_p = _sc_agg()(x_p, src2, dst2, z128)
    s1 = _tc_self(x_p, W1_self, b1.reshape(1, D))
    h1 = _tc1(s1, agg1_p, deg_p, W1_neigh)
    agg2_p = _sc_agg()(h1, src2, dst2, z128)
    s2 = _tc_self(h1, W2_self, b2.reshape(1, D))
    out = _tc2(s2, agg2_p, deg_p, W2_neigh, W3, b3.reshape(1, D))
    return out[:N_NODES]


# speedup vs baseline: 1.0359x; 1.0359x over previous
import functools

import jax
import jax.numpy as jnp
from jax import lax
from jax.experimental import pallas as pl
from jax.experimental.pallas import tpu as pltpu
from jax.experimental.pallas import tpu_sc as plsc

N_NODES = 10000
N_EDGES = 320000
D = 128

NC = 2
NS = 16
NW = NC * NS
LANE = 32

N_PAD = 10112
E_PAD = 327680
ROWS_TOTAL = E_PAD // LANE
ROWS_PW = ROWS_TOTAL // NW
GCH = 8
NBUF = 4
FAST_CID = 1
FAST_ROWS = 448
SLOW_ROWS = 192
STRIPE = N_PAD // NS
DEG_W = 128


def _sc_mesh():
    return plsc.VectorSubcoreMesh(core_axis_name="c", subcore_axis_name="s",
                                  num_cores=NC, num_subcores=NS)


def _zero_stripe(z_hbm, stage, sh, sid):
    pltpu.sync_copy(z_hbm, stage)
    full, rem = divmod(STRIPE, LANE)
    for k in range(full):
        pltpu.sync_copy(stage, sh.at[pl.ds(sid * STRIPE + k * LANE, LANE)])
    if rem:
        pltpu.sync_copy(stage.at[pl.ds(0, rem)],
                        sh.at[pl.ds(sid * STRIPE + full * LANE, rem)])


@functools.cache
def _sc_agg():
    def body(h_hbm, src_hbm, dst_hbm, z_hbm, agg_out,
             idx_v, rows, sg0, sg1, sg2, sg3, ss0, ss1, ss2, ss3, agg_sh):
        cid = lax.axis_index("c")
        sid = lax.axis_index("s")
        wid = cid * NS + sid

        _zero_stripe(z_hbm, rows.at[0], agg_sh, sid)
        plsc.subcore_barrier()

        base = jnp.where(cid == FAST_CID,
                         sid * FAST_ROWS,
                         NS * FAST_ROWS + sid * SLOW_ROWS)
        nchunks = jnp.where(cid == FAST_CID, FAST_ROWS // GCH,
                            SLOW_ROWS // GCH)
        sg = (sg0, sg1, sg2, sg3)
        ss = (ss0, ss1, ss2, ss3)
        LOOK = NBUF - 1

        def chunk(c, carry):
            pltpu.sync_copy(src_hbm.at[pl.ds(base + c * GCH, GCH)],
                            idx_v.at[0])
            pltpu.sync_copy(dst_hbm.at[pl.ds(base + c * GCH, GCH)],
                            idx_v.at[1])

            def gather(r):
                return pltpu.async_copy(h_hbm.at[idx_v.at[0, r]],
                                        rows.at[r % NBUF], sg[r % NBUF])

            gat = {r: gather(r) for r in range(LOOK)}
            sca = {}
            for r in range(GCH):
                gat[r].wait()
                nxt = r + LOOK
                if nxt < GCH:
                    if r >= 1:
                        sca[r - 1].wait()
                    gat[nxt] = gather(nxt)
                sca[r] = pltpu.async_copy(rows.at[r % NBUF],
                                          agg_sh.at[idx_v.at[1, r]],
                                          ss[r % NBUF], add=True)
            for r in range(max(0, GCH - LOOK - 1), GCH):
                sca[r].wait()
            return carry

        lax.fori_loop(0, nchunks, chunk, 0)

        plsc.subcore_barrier()
        pltpu.sync_copy(agg_sh.at[pl.ds(sid * STRIPE, STRIPE)],
                        agg_out.at[cid, pl.ds(sid * STRIPE, STRIPE)])

    return pl.kernel(
        body,
        out_type=jax.ShapeDtypeStruct((NC, N_PAD, D), jnp.float32),
        mesh=_sc_mesh(),
        scratch_types=[
            pltpu.VMEM((2, GCH, LANE), jnp.int32),
            pltpu.VMEM((NBUF, LANE, D), jnp.float32),
            pltpu.SemaphoreType.DMA,
            pltpu.SemaphoreType.DMA,
            pltpu.SemaphoreType.DMA,
            pltpu.SemaphoreType.DMA,
            pltpu.SemaphoreType.DMA,
            pltpu.SemaphoreType.DMA,
            pltpu.SemaphoreType.DMA,
            pltpu.SemaphoreType.DMA,
            pltpu.VMEM_SHARED((N_PAD, D), jnp.float32),
        ],
    )


@functools.cache
def _sc_deg():
    def body(dst_hbm, z_hbm, ones_hbm, deg_out, idx_v, ones_v, sem, deg_sh):
        cid = lax.axis_index("c")
        sid = lax.axis_index("s")
        wid = cid * NS + sid

        _zero_stripe(z_hbm, ones_v, deg_sh, sid)
        pltpu.sync_copy(ones_hbm, ones_v)
        plsc.subcore_barrier()

        base = wid * ROWS_PW

        def chunk(c, carry):
            pltpu.sync_copy(dst_hbm.at[pl.ds(base + c * GCH, GCH)],
                            idx_v.at[0])
            sca = [pltpu.async_copy(ones_v, deg_sh.at[idx_v.at[0, r]],
                                    sem, add=True)
                   for r in range(GCH)]
            for s in sca:
                s.wait()
            return carry

        lax.fori_loop(0, ROWS_PW // GCH, chunk, 0)

        plsc.subcore_barrier()
        pltpu.sync_copy(deg_sh.at[pl.ds(sid * STRIPE, STRIPE)],
                        deg_out.at[cid, pl.ds(sid * STRIPE, STRIPE)])

    return pl.kernel(
        body,
        out_type=jax.ShapeDtypeStruct((NC, N_PAD, DEG_W), jnp.float32),
        mesh=_sc_mesh(),
        scratch_types=[
            pltpu.VMEM((1, GCH, LANE), jnp.int32),
            pltpu.VMEM((LANE, DEG_W), jnp.float32),
            pltpu.SemaphoreType.DMA,
            pltpu.VMEM_SHARED((N_PAD, DEG_W), jnp.float32),
        ],
    )


ROW_BLK = 1264


_DN = (((1,), (1,)), ((), ()))


def _tc_self_body(x_ref, ws_ref, b_ref, out_ref):
    out_ref[...] = lax.dot_general(x_ref[...], ws_ref[...], _DN,
                                   preferred_element_type=jnp.float32) + b_ref[...]


def _tc1_body(s_ref, aggp_ref, degp_ref, wn_ref, out_ref):
    agg = aggp_ref[0] + aggp_ref[1]
    deg = degp_ref[0, :, 0:1] + degp_ref[1, :, 0:1]
    hn = agg / jnp.maximum(deg, 1.0)
    h = s_ref[...] + lax.dot_general(hn, wn_ref[...], _DN,
                                     preferred_element_type=jnp.float32)
    out_ref[...] = jnp.maximum(h, 0.0)


def _tc2_body(s_ref, aggp_ref, degp_ref, wn_ref, w3_ref, b3_ref, out_ref):
    agg = aggp_ref[0] + aggp_ref[1]
    deg = degp_ref[0, :, 0:1] + degp_ref[1, :, 0:1]
    hn = agg / jnp.maximum(deg, 1.0)
    h2 = s_ref[...] + lax.dot_general(hn, wn_ref[...], _DN,
                                      preferred_element_type=jnp.float32)
    out_ref[...] = lax.dot_general(h2, w3_ref[...], _DN,
                                   preferred_element_type=jnp.float32) + b3_ref[...]


def _row_specs():
    row = pl.BlockSpec((ROW_BLK, D), lambda i: (i, 0))
    aggp = pl.BlockSpec((NC, ROW_BLK, D), lambda i: (0, i, 0))
    degp = pl.BlockSpec((NC, ROW_BLK, DEG_W), lambda i: (0, i, 0))
    w = pl.BlockSpec((D, D), lambda i: (0, 0))
    b = pl.BlockSpec((1, D), lambda i: (0, 0))
    return row, aggp, degp, w, b


def _tc_self(x, ws, b):
    row, _, _, w, bias = _row_specs()
    return pl.pallas_call(
        _tc_self_body,
        grid=(N_PAD // ROW_BLK,),
        in_specs=[row, w, bias],
        out_specs=row,
        out_shape=jax.ShapeDtypeStruct((N_PAD, D), jnp.float32),
    )(x, ws, b)


def _tc1(s, aggp, degp, wn):
    row, aggp_s, degp_s, w, _ = _row_specs()
    return pl.pallas_call(
        _tc1_body,
        grid=(N_PAD // ROW_BLK,),
        in_specs=[row, aggp_s, degp_s, w],
        out_specs=row,
        out_shape=jax.ShapeDtypeStruct((N_PAD, D), jnp.float32),
    )(s, aggp, degp, wn)


def _tc2(s, aggp, degp, wn, w3, b3):
    row, aggp_s, degp_s, w, bias = _row_specs()
    return pl.pallas_call(
        _tc2_body,
        grid=(N_PAD // ROW_BLK,),
        in_specs=[row, aggp_s, degp_s, w, w, bias],
        out_specs=row,
        out_shape=jax.ShapeDtypeStruct((N_PAD, D), jnp.float32),
    )(s, aggp, degp, wn, w3, b3)


@jax.jit
def kernel(x, edge_index, W1_self, W1_neigh, b1, W2_self, W2_neigh, b2,
           W3, b3):
    src = edge_index[0].astype(jnp.int32)
    dst = edge_index[1].astype(jnp.int32)
    pad = E_PAD - N_EDGES
    src_p = jnp.concatenate([src, jnp.zeros((pad,), jnp.int32)])
    dst_p = jnp.concatenate([dst, jnp.full((pad,), N_NODES, jnp.int32)])
    src2 = src_p.reshape(ROWS_TOTAL, LANE)
    dst2 = dst_p.reshape(ROWS_TOTAL, LANE)

    x_p = jnp.pad(x, ((0, N_PAD - N_NODES), (0, 0)))
    z128 = jnp.zeros((LANE, D), jnp.float32)
    ones = jnp.ones((LANE, DEG_W), jnp.float32)

    deg_p = _sc_deg()(dst2, z128, ones)
    agg1_p = _sc_agg()(x_p, src2, dst2, z128)
    s1 = _tc_self(x_p, W1_self, b1.reshape(1, D))
    h1 = _tc1(s1, agg1_p, deg_p, W1_neigh)
    agg2_p = _sc_agg()(h1, src2, dst2, z128)
    s2 = _tc_self(h1, W2_self, b2.reshape(1, D))
    out = _tc2(s2, agg2_p, deg_p, W2_neigh, W3, b3.reshape(1, D))
    return out[:N_NODES]
